# baseline (speedup 1.0000x reference)
import functools

import jax
import jax.numpy as jnp
from jax import lax
from jax.experimental import pallas as pl
from jax.experimental.pallas import tpu as pltpu
from jax.experimental.pallas import tpu_sc as plsc

N_ROWS = 1_000_000
N_CLS = 64


TC_BLOCK = 8000
TC_GRID = N_ROWS // TC_BLOCK


def _argmax_body(x_ref, o_ref):
    x = x_ref[...]
    m = jnp.max(x, axis=1, keepdims=True)
    iota = lax.broadcasted_iota(jnp.int32, (TC_BLOCK, N_CLS), 1)
    first = jnp.min(jnp.where(x == m, iota, N_CLS), axis=1)
    o_ref[...] = first.reshape(1, 1, TC_BLOCK)


def _tc_argmax(x):
    return pl.pallas_call(
        _argmax_body,
        grid=(TC_GRID,),
        in_specs=[pl.BlockSpec((TC_BLOCK, N_CLS), lambda i: (i, 0))],
        out_specs=pl.BlockSpec((1, 1, TC_BLOCK), lambda i: (i, 0, 0)),
        out_shape=jax.ShapeDtypeStruct((TC_GRID, 1, TC_BLOCK), jnp.int32),
    )(x)


NSUB = 16
CHUNK = 4000
N_CHUNKS = N_ROWS // CHUNK
GROUPS = CHUNK // 16


def _bincount_body(idx_hbm, out_hbm, cbuf, hist, bins, iota_ref, outv, shared):
    c = lax.axis_index("c")
    s = lax.axis_index("s")
    active = c == 0

    lane = lax.broadcasted_iota(jnp.int32, (16,), 0)
    z16 = jnp.zeros((16,), jnp.int32)
    ones = jnp.ones((16,), jnp.int32)
    laneoff = lane * N_CLS

    for i in range(N_CLS):
        hist[pl.ds(i * 16, 16)] = z16
    for cg in range(4):
        iota_ref[pl.ds(cg * 16, 16)] = lane + cg * 16

    @pl.when(jnp.logical_and(active, s == 0))
    def _():
        pltpu.sync_copy(hist.at[pl.ds(0, N_CLS)], shared)

    plsc.subcore_barrier()

    n_chunks_w = jnp.where(s < (N_CHUNKS % NSUB), N_CHUNKS // NSUB + 1,
                           N_CHUNKS // NSUB)

    @pl.when(active)
    def _():
        def chunk_body(k, _):
            base = (s + k * NSUB) * CHUNK
            pltpu.sync_copy(idx_hbm.at[pl.ds(base, CHUNK)], cbuf)

            def group_body(g, _):
                v = cbuf[pl.ds(g * 16, 16)]
                plsc.addupdate_scatter(hist, [laneoff + v], ones)
                return 0

            lax.fori_loop(0, GROUPS, group_body, 0, unroll=8)
            return 0

        lax.fori_loop(0, n_chunks_w, chunk_body, 0)

        for cg in range(4):
            acc = z16
            for r in range(16):
                acc = acc + hist[pl.ds(r * N_CLS + cg * 16, 16)]
            bins[pl.ds(cg * 16, 16)] = acc

        pltpu.sync_copy(bins, shared.at[iota_ref], add=True)

    plsc.subcore_barrier()

    @pl.when(jnp.logical_and(active, s == 0))
    def _():
        pltpu.sync_copy(shared, bins)
        inv = jnp.float32(1.0 / N_ROWS)
        for cg in range(4):
            cnt = bins[pl.ds(cg * 16, 16)]
            outv[pl.ds(cg * 16, 16)] = cnt.astype(jnp.float32) * inv
        pltpu.sync_copy(outv, out_hbm)


def _sc_bincount(idx):
    mesh = plsc.VectorSubcoreMesh(core_axis_name="c", subcore_axis_name="s")
    f = functools.partial(
        pl.kernel,
        mesh=mesh,
        compiler_params=pltpu.CompilerParams(needs_layout_passes=False),
        out_type=jax.ShapeDtypeStruct((N_CLS,), jnp.float32),
        scratch_types=[
            pltpu.VMEM((CHUNK,), jnp.int32),
            pltpu.VMEM((16 * N_CLS,), jnp.int32),
            pltpu.VMEM((N_CLS,), jnp.int32),
            pltpu.VMEM((N_CLS,), jnp.int32),
            pltpu.VMEM((N_CLS,), jnp.float32),
            pltpu.VMEM_SHARED((N_CLS,), jnp.int32),
        ],
    )(_bincount_body)
    return f(idx)


def kernel(input):
    idx = _tc_argmax(input).reshape(N_ROWS)
    return _sc_bincount(idx)


# speedup vs baseline: 1.5481x; 1.5481x over previous
import functools

import jax
import jax.numpy as jnp
from jax import lax
from jax.experimental import pallas as pl
from jax.experimental.pallas import tpu as pltpu
from jax.experimental.pallas import tpu_sc as plsc

N_ROWS = 1_000_000
N_CLS = 64


TC_BLOCK = 8000
TC_GRID = N_ROWS // TC_BLOCK


TC_CHUNK = 400


def _argmax_body(x_ref, o_ref):
    iota1 = lax.broadcasted_iota(jnp.int32, (1, N_CLS), 1).astype(jnp.float32)
    acc = jnp.zeros((1, N_CLS), jnp.float32)
    for ch in range(TC_BLOCK // TC_CHUNK):
        x = x_ref[pl.ds(ch * TC_CHUNK, TC_CHUNK), :]
        m = jnp.max(x, axis=1, keepdims=True)
        t = jnp.where(x == m, iota1, jnp.float32(N_CLS))
        fi = jnp.min(t, axis=1, keepdims=True)
        oh = jnp.where(iota1 == fi, jnp.float32(1.0), jnp.float32(0.0))
        acc = acc + jnp.sum(oh, axis=0, keepdims=True)
    o_ref[...] = acc.reshape(1, 1, N_CLS)


def _tc_counts(x):
    return pl.pallas_call(
        _argmax_body,
        grid=(TC_GRID,),
        in_specs=[pl.BlockSpec((TC_BLOCK, N_CLS), lambda i: (i, 0))],
        out_specs=pl.BlockSpec((1, 1, N_CLS), lambda i: (i, 0, 0)),
        out_shape=jax.ShapeDtypeStruct((TC_GRID, 1, N_CLS), jnp.float32),
    )(x)


NSUB = 16
NPART = TC_GRID * N_CLS
NPAD = 8192


def _reduce_body(p_hbm, out_hbm, pbuf, bins, iota_ref, outv, shared):
    c = lax.axis_index("c")
    s = lax.axis_index("s")
    active = c == 0

    lane = lax.broadcasted_iota(jnp.int32, (16,), 0)
    zf = jnp.zeros((16,), jnp.float32)

    for cg in range(4):
        iota_ref[pl.ds(cg * 16, 16)] = lane + cg * 16
    for cg in range(4):
        bins[pl.ds(cg * 16, 16)] = zf

    @pl.when(jnp.logical_and(active, s == 0))
    def _():
        pltpu.sync_copy(bins, shared)

    plsc.subcore_barrier()

    @pl.when(active)
    def _():
        pltpu.sync_copy(p_hbm, pbuf.at[pl.ds(0, NPART)])
        for i in range((NPAD - NPART) // 16):
            pbuf[pl.ds(NPART + i * 16, 16)] = zf
        for cg in range(4):
            acc = zf
            for k in range(8):
                acc = acc + pbuf[pl.ds((s + 16 * k) * N_CLS + cg * 16, 16)]
            bins[pl.ds(cg * 16, 16)] = acc
        pltpu.sync_copy(bins, shared.at[iota_ref], add=True)

    plsc.subcore_barrier()

    @pl.when(jnp.logical_and(active, s == 0))
    def _():
        pltpu.sync_copy(shared, bins)
        inv = jnp.float32(1.0 / N_ROWS)
        for cg in range(4):
            outv[pl.ds(cg * 16, 16)] = bins[pl.ds(cg * 16, 16)] * inv
        pltpu.sync_copy(outv, out_hbm)


def _sc_reduce(partials):
    mesh = plsc.VectorSubcoreMesh(core_axis_name="c", subcore_axis_name="s")
    f = functools.partial(
        pl.kernel,
        mesh=mesh,
        compiler_params=pltpu.CompilerParams(needs_layout_passes=False),
        out_type=jax.ShapeDtypeStruct((N_CLS,), jnp.float32),
        scratch_types=[
            pltpu.VMEM((NPAD,), jnp.float32),
            pltpu.VMEM((N_CLS,), jnp.float32),
            pltpu.VMEM((N_CLS,), jnp.int32),
            pltpu.VMEM((N_CLS,), jnp.float32),
            pltpu.VMEM_SHARED((N_CLS,), jnp.float32),
        ],
    )(_reduce_body)
    return f(partials)


def kernel(input):
    partials = _tc_counts(input).reshape(NPART)
    return _sc_reduce(partials)
